# baseline (speedup 1.0000x reference)
import functools

import jax
import jax.numpy as jnp
from jax import lax
from jax.experimental import pallas as pl
from jax.experimental.pallas import tpu as pltpu
from jax.experimental.pallas import tpu_sc as plsc

V = 1000000
D = 64
NW = 32
NBLK = V // 128
REM = V - NBLK * 128


def _relayout_kernel(tt_hbm, out_hbm, src_v, dst_v, rsrc_v, rdst_v,
                     gsem, wsem):
    c = lax.axis_index("c")
    s = lax.axis_index("s")
    wid = s * 2 + c
    iota = lax.iota(jnp.int32, 16)
    kmax = NBLK // NW + 1

    def blk(k):
        return k * NW + wid

    def gather(b, p):
        pltpu.async_copy(
            tt_hbm.at[:, pl.ds(b * 128, 128)], src_v.at[p], gsem.at[p])

    def wait_gather(b, p):
        pltpu.make_async_copy(
            tt_hbm.at[:, pl.ds(b * 128, 128)], src_v.at[p], gsem.at[p]).wait()

    def write(b, p):
        pltpu.async_copy(
            dst_v.at[p], out_hbm.at[pl.ds(b * 64, 64)], wsem.at[p])

    def wait_write(b, p):
        pltpu.make_async_copy(
            dst_v.at[p], out_hbm.at[pl.ds(b * 64, 64)], wsem.at[p]).wait()

    def transpose(p):
        def body(r, _):
            for m in range(8):
                cvec = 16 * (m % 4) + iota
                i = 2 * r + (1 if m >= 4 else 0)
                val = plsc.load_gather(
                    src_v.at[p], [cvec, jnp.full((16,), 0, jnp.int32) + i])
                dst_v[p, r, pl.ds(16 * m, 16)] = val
            return 0

        lax.fori_loop(0, 64, body, 0, unroll=False)

    gather(blk(0), 0)

    def step(k, _):
        for p in range(2):
            kk = k * 2 + p

            @pl.when(blk(kk) < NBLK)
            def _():
                @pl.when(blk(kk + 1) < NBLK)
                def _():
                    gather(blk(kk + 1), 1 - p)
                wait_gather(blk(kk), p)

                @pl.when(kk >= 2)
                def _():
                    wait_write(blk(kk - 2), p)
                transpose(p)
                write(blk(kk), p)
        return 0

    lax.fori_loop(0, (kmax + 1) // 2, step, 0, unroll=False)

    for p in range(2):
        last = kmax - 2 + p

        @pl.when(blk(last) < NBLK)
        def _():
            wait_write(blk(last), last % 2)

    @pl.when(wid == 0)
    def _():
        pltpu.sync_copy(tt_hbm.at[:, pl.ds(NBLK * 128, REM)], rsrc_v)

        def body(r, _):
            for m in range(8):
                cvec = 16 * (m % 4) + iota
                i = 2 * r + (1 if m >= 4 else 0)
                val = plsc.load_gather(
                    rsrc_v, [cvec, jnp.full((16,), 0, jnp.int32) + i])
                rdst_v[r, pl.ds(16 * m, 16)] = val
            return 0

        lax.fori_loop(0, REM // 2, body, 0, unroll=False)
        pltpu.sync_copy(rdst_v, out_hbm.at[pl.ds(NBLK * 64, REM // 2)])


def _gather_kernel(bpw: int, seq: int, table_hbm, idx_hbm, out_hbm,
                   idx_v, idxt_v, rows_v, tile_v, gsem, wsem):
    c = lax.axis_index("c")
    s = lax.axis_index("s")
    wid = s * 2 + c
    iota = lax.iota(jnp.int32, 16)

    pltpu.sync_copy(idx_hbm.at[pl.ds(wid * bpw * seq, bpw * seq)], idx_v)

    def tbody(t, _):
        for h in range(bpw // 16):
            addr = (16 * h + iota) * seq + t
            val = plsc.load_gather(idx_v, [addr])
            idxt_v[t, pl.ds(16 * h, 16)] = val
        return 0

    lax.fori_loop(0, seq, tbody, 0, unroll=False)

    def gather(t, p):
        pltpu.async_copy(
            table_hbm.at[idxt_v.at[t]], rows_v.at[p], gsem.at[p])

    def wait_gather(t, p):
        pltpu.make_async_copy(
            table_hbm.at[idxt_v.at[t]], rows_v.at[p], gsem.at[p]).wait()

    def write(t, p):
        pltpu.async_copy(tile_v.at[p], out_hbm.at[t, :, wid], wsem.at[p])

    def wait_write(t, p):
        pltpu.make_async_copy(
            tile_v.at[p], out_hbm.at[t, :, wid], wsem.at[p]).wait()

    def transpose(p):
        def body(h, _):
            bvec = 16 * h + iota
            for c8 in range(8):
                for ci in range(8):
                    val = plsc.load_gather(
                        rows_v.at[p],
                        [bvec, jnp.full((16,), 8 * c8 + ci, jnp.int32)])
                    tile_v[p, c8, ci, pl.ds(16 * h, 16)] = val
            return 0

        lax.fori_loop(0, 8, body, 0, unroll=False)

    gather(0, 0)

    def step(g, _):
        for p in range(2):
            t = g * 2 + p

            @pl.when(t + 1 < seq)
            def _():
                gather(t + 1, 1 - p)
            wait_gather(t, p)

            @pl.when(t >= 2)
            def _():
                wait_write(t - 2, p)
            transpose(p)
            write(t, p)
        return 0

    lax.fori_loop(0, seq // 2, step, 0, unroll=False)
    wait_write(seq - 2, 0)
    wait_write(seq - 1, 1)


@jax.jit
def kernel(indices, table):
    batch, seq = indices.shape
    bpw = batch // NW
    mesh = plsc.VectorSubcoreMesh(core_axis_name="c", subcore_axis_name="s")

    tlin2 = pl.kernel(
        _relayout_kernel,
        mesh=mesh,
        out_type=jax.ShapeDtypeStruct((V // 2, 128), jnp.float32),
        compiler_params=pltpu.CompilerParams(use_tc_tiling_on_sc=True, needs_layout_passes=False),
        scratch_types=[
            pltpu.VMEM((2, D, 128), jnp.float32),
            pltpu.VMEM((2, D, 128), jnp.float32),
            pltpu.VMEM((D, REM), jnp.float32),
            pltpu.VMEM((REM // 2, 128), jnp.float32),
            pltpu.SemaphoreType.DMA((2,)),
            pltpu.SemaphoreType.DMA((2,)),
        ],
    )(table.T)
    tlin = tlin2.reshape(V, D)

    idx = indices.reshape(-1).astype(jnp.int32)
    out5 = pl.kernel(
        functools.partial(_gather_kernel, bpw, seq),
        mesh=mesh,
        out_type=jax.ShapeDtypeStruct((seq, 8, NW, 8, 128), jnp.float32),
        compiler_params=pltpu.CompilerParams(use_tc_tiling_on_sc=False, needs_layout_passes=False),
        scratch_types=[
            pltpu.VMEM((bpw * seq,), jnp.int32),
            pltpu.VMEM((seq, bpw), jnp.int32),
            pltpu.VMEM((2, bpw, D), jnp.float32),
            pltpu.VMEM((2, 8, 8, 128), jnp.float32),
            pltpu.SemaphoreType.DMA((2,)),
            pltpu.SemaphoreType.DMA((2,)),
        ],
    )(tlin, idx)
    return out5.transpose(2, 4, 0, 1, 3).reshape(batch, seq, D)


# speedup vs baseline: 1.4201x; 1.4201x over previous
import functools

import jax
import jax.numpy as jnp
from jax import lax
from jax.experimental import pallas as pl
from jax.experimental.pallas import tpu as pltpu
from jax.experimental.pallas import tpu_sc as plsc

V = 1000000
D = 64
NW = 32
NBLK = V // 128
REM = V - NBLK * 128


def _relayout_kernel(tt_hbm, out_hbm, src_v, dst_v, rsrc_v, rdst_v,
                     gsem, wsem):
    c = lax.axis_index("c")
    s = lax.axis_index("s")
    wid = s * 2 + c
    iota = lax.iota(jnp.int32, 16)
    kmax = NBLK // NW + 1

    def blk(k):
        return k * NW + wid

    def gather(b, p):
        pltpu.async_copy(
            tt_hbm.at[:, pl.ds(b * 128, 128)],
            src_v.at[p, :, pl.ds(0, 128)], gsem.at[p])

    def wait_gather(b, p):
        pltpu.make_async_copy(
            tt_hbm.at[:, pl.ds(b * 128, 128)],
            src_v.at[p, :, pl.ds(0, 128)], gsem.at[p]).wait()

    def write(b, p):
        pltpu.async_copy(
            dst_v.at[p, :, pl.ds(0, 128)],
            out_hbm.at[pl.ds(b * 64, 64)], wsem.at[p])

    def wait_write(b, p):
        pltpu.make_async_copy(
            dst_v.at[p, :, pl.ds(0, 128)],
            out_hbm.at[pl.ds(b * 64, 64)], wsem.at[p]).wait()

    cvecs = [16 * g + iota for g in range(4)]

    def transpose(p):
        def body(i, _):
            ivec = jnp.full((16,), 0, jnp.int32) + i
            r = i >> 1
            colbase = (i & 1) * 64
            for g in range(4):
                val = plsc.load_gather(src_v.at[p], [cvecs[g], ivec])
                dst_v[p, r, pl.ds(colbase + 16 * g, 16)] = val
            return 0

        lax.fori_loop(0, 128, body, 0, unroll=False)

    gather(blk(0), 0)

    def step(k, _):
        for p in range(2):
            kk = k * 2 + p

            @pl.when(blk(kk) < NBLK)
            def _():
                @pl.when(blk(kk + 1) < NBLK)
                def _():
                    gather(blk(kk + 1), 1 - p)
                wait_gather(blk(kk), p)

                @pl.when(kk >= 2)
                def _():
                    wait_write(blk(kk - 2), p)
                transpose(p)
                write(blk(kk), p)
        return 0

    lax.fori_loop(0, (kmax + 1) // 2, step, 0, unroll=False)

    for p in range(2):
        last = kmax - 2 + p

        @pl.when(blk(last) < NBLK)
        def _():
            wait_write(blk(last), last % 2)

    @pl.when(wid == 0)
    def _():
        pltpu.sync_copy(tt_hbm.at[:, pl.ds(NBLK * 128, REM)], rsrc_v)

        def body(r, _):
            for m in range(8):
                cvec = 16 * (m % 4) + iota
                i = 2 * r + (1 if m >= 4 else 0)
                val = plsc.load_gather(
                    rsrc_v, [cvec, jnp.full((16,), 0, jnp.int32) + i])
                rdst_v[r, pl.ds(16 * m, 16)] = val
            return 0

        lax.fori_loop(0, REM // 2, body, 0, unroll=False)
        pltpu.sync_copy(rdst_v, out_hbm.at[pl.ds(NBLK * 64, REM // 2)])


def _gather_kernel(bpw: int, seq: int, table_hbm, idx_hbm, out_hbm,
                   idx_v, idxt_v, rows_v, tile_v, gsem, wsem):
    c = lax.axis_index("c")
    s = lax.axis_index("s")
    wid = s * 2 + c
    iota = lax.iota(jnp.int32, 16)

    pltpu.sync_copy(idx_hbm.at[pl.ds(wid * bpw * seq, bpw * seq)], idx_v)

    def tbody(t, _):
        for h in range(bpw // 16):
            addr = (16 * h + iota) * seq + t
            val = plsc.load_gather(idx_v, [addr])
            idxt_v[t, pl.ds(16 * h, 16)] = val
        return 0

    lax.fori_loop(0, seq, tbody, 0, unroll=False)

    def gather(t, p):
        pltpu.async_copy(
            table_hbm.at[idxt_v.at[t]], rows_v.at[p], gsem.at[p])

    def wait_gather(t, p):
        pltpu.make_async_copy(
            table_hbm.at[idxt_v.at[t]], rows_v.at[p], gsem.at[p]).wait()

    def write(t, p):
        pltpu.async_copy(
            tile_v.at[p, :, :, pl.ds(0, 128)],
            out_hbm.at[t, :, wid], wsem.at[p])

    def wait_write(t, p):
        pltpu.make_async_copy(
            tile_v.at[p, :, :, pl.ds(0, 128)],
            out_hbm.at[t, :, wid], wsem.at[p]).wait()

    c8vecs = [(16 * g + iota) >> 3 for g in range(4)]
    civecs = [iota & 7 for _ in range(4)]

    def transpose(p):
        def body(b, _):
            bvec = jnp.full((16,), 0, jnp.int32) + b
            for g in range(4):
                val = rows_v[p, b, pl.ds(16 * g, 16)]
                plsc.store_scatter(
                    tile_v.at[p], [c8vecs[g], civecs[g], bvec], val)
            return 0

        lax.fori_loop(0, 128, body, 0, unroll=False)

    gather(0, 0)

    def step(g, _):
        for p in range(2):
            t = g * 2 + p

            @pl.when(t + 1 < seq)
            def _():
                gather(t + 1, 1 - p)
            wait_gather(t, p)

            @pl.when(t >= 2)
            def _():
                wait_write(t - 2, p)
            transpose(p)
            write(t, p)
        return 0

    lax.fori_loop(0, seq // 2, step, 0, unroll=False)
    wait_write(seq - 2, 0)
    wait_write(seq - 1, 1)


@jax.jit
def kernel(indices, table):
    batch, seq = indices.shape
    bpw = batch // NW
    mesh = plsc.VectorSubcoreMesh(core_axis_name="c", subcore_axis_name="s")

    tlin2 = pl.kernel(
        _relayout_kernel,
        mesh=mesh,
        out_type=jax.ShapeDtypeStruct((V // 2, 128), jnp.float32),
        compiler_params=pltpu.CompilerParams(use_tc_tiling_on_sc=True, needs_layout_passes=False),
        scratch_types=[
            pltpu.VMEM((2, D, 129), jnp.float32),
            pltpu.VMEM((2, D, 129), jnp.float32),
            pltpu.VMEM((D, REM), jnp.float32),
            pltpu.VMEM((REM // 2, 128), jnp.float32),
            pltpu.SemaphoreType.DMA((2,)),
            pltpu.SemaphoreType.DMA((2,)),
        ],
    )(table.T)
    tlin = tlin2.reshape(V, D)

    idx = indices.reshape(-1).astype(jnp.int32)
    out5 = pl.kernel(
        functools.partial(_gather_kernel, bpw, seq),
        mesh=mesh,
        out_type=jax.ShapeDtypeStruct((seq, 8, NW, 8, 128), jnp.float32),
        compiler_params=pltpu.CompilerParams(use_tc_tiling_on_sc=False, needs_layout_passes=False),
        scratch_types=[
            pltpu.VMEM((bpw * seq,), jnp.int32),
            pltpu.VMEM((seq, bpw), jnp.int32),
            pltpu.VMEM((2, bpw, D), jnp.float32),
            pltpu.VMEM((2, 8, 8, 133), jnp.float32),
            pltpu.SemaphoreType.DMA((2,)),
            pltpu.SemaphoreType.DMA((2,)),
        ],
    )(tlin, idx)
    return out5.transpose(2, 4, 0, 1, 3).reshape(batch, seq, D)


# speedup vs baseline: 2.8498x; 2.0068x over previous
import functools

import jax
import jax.numpy as jnp
from jax import lax
from jax.experimental import pallas as pl
from jax.experimental.pallas import tpu as pltpu
from jax.experimental.pallas import tpu_sc as plsc

V = 1000000
D = 64
NW = 32
NBLK = V // 128
REM = V - NBLK * 128


def _relayout_kernel(tt_hbm, out_hbm, src_v, dst_v, rsrc_v, rdst_v,
                     gsem, wsem):
    c = lax.axis_index("c")
    s = lax.axis_index("s")
    wid = s * 2 + c
    iota = lax.iota(jnp.int32, 16)
    kmax = NBLK // NW + 1

    def blk(k):
        return k * NW + wid

    def gather(b, p):
        pltpu.async_copy(
            tt_hbm.at[:, pl.ds(b * 128, 128)], src_v.at[p], gsem.at[p])

    def wait_gather(b, p):
        pltpu.make_async_copy(
            tt_hbm.at[:, pl.ds(b * 128, 128)], src_v.at[p], gsem.at[p]).wait()

    def write(b, p):
        pltpu.async_copy(
            dst_v.at[p], out_hbm.at[pl.ds(b * 64, 64)], wsem.at[p])

    def wait_write(b, p):
        pltpu.make_async_copy(
            dst_v.at[p], out_hbm.at[pl.ds(b * 64, 64)], wsem.at[p]).wait()

    ivecs = [16 * u + iota for u in range(8)]

    def transpose(p):
        def body(base, _):
            cvec = (base + iota) & 63
            for u in range(8):
                q = ivecs[u] * 64 + cvec
                val = plsc.load_gather(src_v.at[p], [cvec, ivecs[u]])
                plsc.store_scatter(dst_v.at[p], [q >> 7, q & 127], val)
            return 0

        lax.fori_loop(0, 64, body, 0, unroll=False)

    gather(blk(0), 0)

    def step(k, _):
        for p in range(2):
            kk = k * 2 + p

            @pl.when(blk(kk) < NBLK)
            def _():
                @pl.when(blk(kk + 1) < NBLK)
                def _():
                    gather(blk(kk + 1), 1 - p)
                wait_gather(blk(kk), p)

                @pl.when(kk >= 2)
                def _():
                    wait_write(blk(kk - 2), p)
                transpose(p)
                write(blk(kk), p)
        return 0

    lax.fori_loop(0, (kmax + 1) // 2, step, 0, unroll=False)

    for p in range(2):
        last = kmax - 2 + p

        @pl.when(blk(last) < NBLK)
        def _():
            wait_write(blk(last), last % 2)

    @pl.when(wid == 0)
    def _():
        pltpu.sync_copy(tt_hbm.at[:, pl.ds(NBLK * 128, REM)], rsrc_v)

        def body(r, _):
            for m in range(8):
                cvec = 16 * (m % 4) + iota
                i = 2 * r + (1 if m >= 4 else 0)
                val = plsc.load_gather(
                    rsrc_v, [cvec, jnp.full((16,), 0, jnp.int32) + i])
                rdst_v[r, pl.ds(16 * m, 16)] = val
            return 0

        lax.fori_loop(0, REM // 2, body, 0, unroll=False)
        pltpu.sync_copy(rdst_v, out_hbm.at[pl.ds(NBLK * 64, REM // 2)])


def _gather_kernel(bpw: int, seq: int, table_hbm, idx_hbm, out_hbm,
                   idx_v, idxt_v, rows_v, tile_v, gsem, wsem):
    c = lax.axis_index("c")
    s = lax.axis_index("s")
    wid = s * 2 + c
    iota = lax.iota(jnp.int32, 16)

    pltpu.sync_copy(idx_hbm.at[pl.ds(wid * bpw * seq, bpw * seq)], idx_v)

    def tbody(t, _):
        for h in range(bpw // 16):
            addr = (16 * h + iota) * seq + t
            val = plsc.load_gather(idx_v, [addr])
            idxt_v[t, pl.ds(16 * h, 16)] = val
        return 0

    lax.fori_loop(0, seq, tbody, 0, unroll=False)

    def gather(t, p):
        pltpu.async_copy(
            table_hbm.at[idxt_v.at[t]], rows_v.at[p], gsem.at[p])

    def wait_gather(t, p):
        pltpu.make_async_copy(
            table_hbm.at[idxt_v.at[t]], rows_v.at[p], gsem.at[p]).wait()

    def write(t, p):
        pltpu.async_copy(tile_v.at[p], out_hbm.at[t, :, wid], wsem.at[p])

    def wait_write(t, p):
        pltpu.make_async_copy(
            tile_v.at[p], out_hbm.at[t, :, wid], wsem.at[p]).wait()

    bvecs = [16 * h + iota for h in range(8)]

    def transpose(p):
        def body(base, _):
            cvec = (base + iota) & 63
            c8v = cvec >> 3
            civ = cvec & 7
            for h in range(8):
                val = plsc.load_gather(rows_v.at[p], [bvecs[h], cvec])
                plsc.store_scatter(tile_v.at[p], [c8v, civ, bvecs[h]], val)
            return 0

        lax.fori_loop(0, 64, body, 0, unroll=False)

    gather(0, 0)

    def step(g, _):
        for p in range(2):
            t = g * 2 + p

            @pl.when(t + 1 < seq)
            def _():
                gather(t + 1, 1 - p)
            wait_gather(t, p)

            @pl.when(t >= 2)
            def _():
                wait_write(t - 2, p)
            transpose(p)
            write(t, p)
        return 0

    lax.fori_loop(0, seq // 2, step, 0, unroll=False)
    wait_write(seq - 2, 0)
    wait_write(seq - 1, 1)


@jax.jit
def kernel(indices, table):
    batch, seq = indices.shape
    bpw = batch // NW
    mesh = plsc.VectorSubcoreMesh(core_axis_name="c", subcore_axis_name="s")

    tlin2 = pl.kernel(
        _relayout_kernel,
        mesh=mesh,
        out_type=jax.ShapeDtypeStruct((V // 2, 128), jnp.float32),
        compiler_params=pltpu.CompilerParams(use_tc_tiling_on_sc=True, needs_layout_passes=False),
        scratch_types=[
            pltpu.VMEM((2, D, 128), jnp.float32),
            pltpu.VMEM((2, D, 128), jnp.float32),
            pltpu.VMEM((D, REM), jnp.float32),
            pltpu.VMEM((REM // 2, 128), jnp.float32),
            pltpu.SemaphoreType.DMA((2,)),
            pltpu.SemaphoreType.DMA((2,)),
        ],
    )(table.T)
    tlin = tlin2.reshape(V, D)

    idx = indices.reshape(-1).astype(jnp.int32)
    out5 = pl.kernel(
        functools.partial(_gather_kernel, bpw, seq),
        mesh=mesh,
        out_type=jax.ShapeDtypeStruct((seq, 8, NW, 8, 128), jnp.float32),
        compiler_params=pltpu.CompilerParams(use_tc_tiling_on_sc=False, needs_layout_passes=False),
        scratch_types=[
            pltpu.VMEM((bpw * seq,), jnp.int32),
            pltpu.VMEM((seq, bpw), jnp.int32),
            pltpu.VMEM((2, bpw, D), jnp.float32),
            pltpu.VMEM((2, 8, 8, 128), jnp.float32),
            pltpu.SemaphoreType.DMA((2,)),
            pltpu.SemaphoreType.DMA((2,)),
        ],
    )(tlin, idx)
    return out5.transpose(2, 4, 0, 1, 3).reshape(batch, seq, D)


# speedup vs baseline: 2.9270x; 1.0271x over previous
import functools

import jax
import jax.numpy as jnp
from jax import lax
from jax.experimental import pallas as pl
from jax.experimental.pallas import tpu as pltpu
from jax.experimental.pallas import tpu_sc as plsc

V = 1000000
D = 64
NW = 32
NBLK = V // 128
REM = V - NBLK * 128


def _relayout_kernel(tt_hbm, out_hbm, src_v, dst_v, rsrc_v, rdst_v,
                     gsem, wsem):
    c = lax.axis_index("c")
    s = lax.axis_index("s")
    wid = s * 2 + c
    iota = lax.iota(jnp.int32, 16)
    kmax = NBLK // NW + 1

    def blk(k):
        return k * NW + wid

    def gather(b, p):
        pltpu.async_copy(
            tt_hbm.at[:, pl.ds(b * 128, 128)], src_v.at[p], gsem.at[p])

    def wait_gather(b, p):
        pltpu.make_async_copy(
            tt_hbm.at[:, pl.ds(b * 128, 128)], src_v.at[p], gsem.at[p]).wait()

    def write(b, p):
        pltpu.async_copy(
            dst_v.at[p], out_hbm.at[pl.ds(b * 64, 64)], wsem.at[p])

    def wait_write(b, p):
        pltpu.make_async_copy(
            dst_v.at[p], out_hbm.at[pl.ds(b * 64, 64)], wsem.at[p]).wait()

    ivecs = [16 * u + iota for u in range(8)]

    def transpose(p):
        def body(base, _):
            cvec = (base + iota) & 63
            for u in range(8):
                q = ivecs[u] * 64 + cvec
                val = plsc.load_gather(src_v.at[p], [cvec, ivecs[u]])
                plsc.store_scatter(dst_v.at[p], [q >> 7, q & 127], val)
            return 0

        lax.fori_loop(0, 64, body, 0, unroll=4)

    gather(blk(0), 0)

    def step(k, _):
        for p in range(2):
            kk = k * 2 + p

            @pl.when(blk(kk) < NBLK)
            def _():
                @pl.when(blk(kk + 1) < NBLK)
                def _():
                    gather(blk(kk + 1), 1 - p)
                wait_gather(blk(kk), p)

                @pl.when(kk >= 2)
                def _():
                    wait_write(blk(kk - 2), p)
                transpose(p)
                write(blk(kk), p)
        return 0

    lax.fori_loop(0, (kmax + 1) // 2, step, 0, unroll=False)

    for p in range(2):
        last = kmax - 2 + p

        @pl.when(blk(last) < NBLK)
        def _():
            wait_write(blk(last), last % 2)

    @pl.when(wid == 0)
    def _():
        pltpu.sync_copy(tt_hbm.at[:, pl.ds(NBLK * 128, REM)], rsrc_v)

        def body(r, _):
            for m in range(8):
                cvec = 16 * (m % 4) + iota
                i = 2 * r + (1 if m >= 4 else 0)
                val = plsc.load_gather(
                    rsrc_v, [cvec, jnp.full((16,), 0, jnp.int32) + i])
                rdst_v[r, pl.ds(16 * m, 16)] = val
            return 0

        lax.fori_loop(0, REM // 2, body, 0, unroll=False)
        pltpu.sync_copy(rdst_v, out_hbm.at[pl.ds(NBLK * 64, REM // 2)])


def _gather_kernel(bpw: int, seq: int, table_hbm, idx_hbm, out_hbm,
                   idx_v, idxt_v, rows_v, tile_v, gsem, wsem):
    c = lax.axis_index("c")
    s = lax.axis_index("s")
    wid = s * 2 + c
    iota = lax.iota(jnp.int32, 16)

    pltpu.sync_copy(idx_hbm.at[pl.ds(wid * bpw * seq, bpw * seq)], idx_v)

    def tbody(t, _):
        for h in range(bpw // 16):
            addr = (16 * h + iota) * seq + t
            val = plsc.load_gather(idx_v, [addr])
            idxt_v[t, pl.ds(16 * h, 16)] = val
        return 0

    lax.fori_loop(0, seq, tbody, 0, unroll=False)

    def gather(t, p):
        pltpu.async_copy(
            table_hbm.at[idxt_v.at[t]], rows_v.at[p], gsem.at[p])

    def wait_gather(t, p):
        pltpu.make_async_copy(
            table_hbm.at[idxt_v.at[t]], rows_v.at[p], gsem.at[p]).wait()

    def write(t, p):
        pltpu.async_copy(tile_v.at[p], out_hbm.at[t, :, wid], wsem.at[p])

    def wait_write(t, p):
        pltpu.make_async_copy(
            tile_v.at[p], out_hbm.at[t, :, wid], wsem.at[p]).wait()

    bvecs = [16 * h + iota for h in range(8)]

    def transpose(p):
        def body(base, _):
            cvec = (base + iota) & 63
            c8v = cvec >> 3
            civ = cvec & 7
            for h in range(8):
                val = plsc.load_gather(rows_v.at[p], [bvecs[h], cvec])
                plsc.store_scatter(tile_v.at[p], [c8v, civ, bvecs[h]], val)
            return 0

        lax.fori_loop(0, 64, body, 0, unroll=4)

    gather(0, 0)

    def step(g, _):
        for p in range(2):
            t = g * 2 + p

            @pl.when(t + 1 < seq)
            def _():
                gather(t + 1, 1 - p)
            wait_gather(t, p)

            @pl.when(t >= 2)
            def _():
                wait_write(t - 2, p)
            transpose(p)
            write(t, p)
        return 0

    lax.fori_loop(0, seq // 2, step, 0, unroll=False)
    wait_write(seq - 2, 0)
    wait_write(seq - 1, 1)


@jax.jit
def kernel(indices, table):
    batch, seq = indices.shape
    bpw = batch // NW
    mesh = plsc.VectorSubcoreMesh(core_axis_name="c", subcore_axis_name="s")

    tlin2 = pl.kernel(
        _relayout_kernel,
        mesh=mesh,
        out_type=jax.ShapeDtypeStruct((V // 2, 128), jnp.float32),
        compiler_params=pltpu.CompilerParams(use_tc_tiling_on_sc=True, needs_layout_passes=False),
        scratch_types=[
            pltpu.VMEM((2, D, 128), jnp.float32),
            pltpu.VMEM((2, D, 128), jnp.float32),
            pltpu.VMEM((D, REM), jnp.float32),
            pltpu.VMEM((REM // 2, 128), jnp.float32),
            pltpu.SemaphoreType.DMA((2,)),
            pltpu.SemaphoreType.DMA((2,)),
        ],
    )(table.T)
    tlin = tlin2.reshape(V, D)

    idx = indices.reshape(-1).astype(jnp.int32)
    out5 = pl.kernel(
        functools.partial(_gather_kernel, bpw, seq),
        mesh=mesh,
        out_type=jax.ShapeDtypeStruct((seq, 8, NW, 8, 128), jnp.float32),
        compiler_params=pltpu.CompilerParams(use_tc_tiling_on_sc=False, needs_layout_passes=False),
        scratch_types=[
            pltpu.VMEM((bpw * seq,), jnp.int32),
            pltpu.VMEM((seq, bpw), jnp.int32),
            pltpu.VMEM((2, bpw, D), jnp.float32),
            pltpu.VMEM((2, 8, 8, 128), jnp.float32),
            pltpu.SemaphoreType.DMA((2,)),
            pltpu.SemaphoreType.DMA((2,)),
        ],
    )(tlin, idx)
    return out5.transpose(2, 4, 0, 1, 3).reshape(batch, seq, D)
